# Initial kernel scaffold; baseline (speedup 1.0000x reference)
import jax
import jax.numpy as jnp
from jax.experimental import pallas as pl


def kernel(node_feature, global_feature, edge_index, attack_edge_index, ally_indices, W_msg, b_msg, W_u1, b_u1, W_u2, b_u2, ln_g, ln_b, W_glob, b_glob, W_m1, b_m1, W_m2, b_m2, W_h1, b_h1, W_h2, b_h2, W_a1, b_a1, W_a2, b_a2):
    raise NotImplementedError("write your pallas kernel here")


# speedup vs baseline: 6.0602x; 6.0602x over previous
import functools

import jax
import jax.numpy as jnp
from jax import lax
from jax.experimental import pallas as pl
from jax.experimental.pallas import tpu as pltpu
from jax.experimental.pallas import tpu_sc as plsc

_NC = 2
_NS = 16
_NW = _NC * _NS
_LANES = 16
_CHUNK = 128


def _msg_body(x_ref, w_ref, b_ref, o_ref):
    m = jnp.tanh(
        jnp.dot(x_ref[...], w_ref[...], preferred_element_type=jnp.float32)
        + b_ref[...])
    o_ref[...] = jnp.maximum(m, 0.0)


def _tc_msg(x, w, b, bn):
    n, d = x.shape
    return pl.pallas_call(
        _msg_body,
        grid=(n // bn,),
        in_specs=[
            pl.BlockSpec((bn, d), lambda i: (i, 0)),
            pl.BlockSpec((d, d), lambda i: (0, 0)),
            pl.BlockSpec((1, d), lambda i: (0, 0)),
        ],
        out_specs=pl.BlockSpec((bn, d), lambda i: (i, 0)),
        out_shape=jax.ShapeDtypeStruct((n, d), jnp.float32),
    )(x, w, b.reshape(1, d))


def _upd_body(with_psum, x_ref, a0_ref, a1_ref, w1x_ref, w1a_ref, b1_ref,
              w2_ref, b2_ref, g_ref, be_ref, *out_refs):
    x = x_ref[...]
    agg = a0_ref[...] + a1_ref[...]
    h = jnp.tanh(
        jnp.dot(x, w1x_ref[...], preferred_element_type=jnp.float32)
        + jnp.dot(agg, w1a_ref[...], preferred_element_type=jnp.float32)
        + b1_ref[...])
    h = jnp.dot(h, w2_ref[...], preferred_element_type=jnp.float32) + b2_ref[...]
    mu = jnp.mean(h, axis=-1, keepdims=True)
    c = h - mu
    var = jnp.mean(c * c, axis=-1, keepdims=True)
    xn = g_ref[...] * c * lax.rsqrt(var + 1e-5) + be_ref[...]
    out_refs[0][...] = xn
    if with_psum:
        out_refs[1][...] = jnp.sum(xn, axis=0)[None, None, :]


def _tc_update(x, a0, a1, w1x, w1a, b1, w2, b2, ln_g, ln_b, bn, with_psum):
    n, d = x.shape
    grid = n // bn
    out_shape = [jax.ShapeDtypeStruct((n, d), jnp.float32)]
    out_specs = [pl.BlockSpec((bn, d), lambda i: (i, 0))]
    if with_psum:
        out_shape.append(jax.ShapeDtypeStruct((grid, 1, d), jnp.float32))
        out_specs.append(pl.BlockSpec((1, 1, d), lambda i: (i, 0, 0)))
    res = pl.pallas_call(
        functools.partial(_upd_body, with_psum),
        grid=(grid,),
        in_specs=[
            pl.BlockSpec((bn, d), lambda i: (i, 0)),
            pl.BlockSpec((bn, d), lambda i: (i, 0)),
            pl.BlockSpec((bn, d), lambda i: (i, 0)),
            pl.BlockSpec((d, d), lambda i: (0, 0)),
            pl.BlockSpec((d, d), lambda i: (0, 0)),
            pl.BlockSpec((1, d), lambda i: (0, 0)),
            pl.BlockSpec((d, d), lambda i: (0, 0)),
            pl.BlockSpec((1, d), lambda i: (0, 0)),
            pl.BlockSpec((1, d), lambda i: (0, 0)),
            pl.BlockSpec((1, d), lambda i: (0, 0)),
        ],
        out_specs=out_specs,
        out_shape=out_shape,
    )(x, a0, a1, w1x, w1a, b1.reshape(1, d), w2, b2.reshape(1, d),
      ln_g.reshape(1, d), ln_b.reshape(1, d))
    if with_psum:
        return res[0], res[1]
    return res[0], None


def _heads_body(n_total, x_ref, ps_ref, gf_ref, wg_ref, bg_ref,
                wm1x_ref, wm1g_ref, bm1_ref, wh1x_ref, wh1g_ref, bh1_ref,
                wa1x_ref, wa1g_ref, ba1_ref, w2c_ref, b2c_ref, o_ref):
    total = jnp.sum(ps_ref[...], axis=(0, 1))[None, :] * (1.0 / n_total)
    g = jnp.tanh(
        jnp.dot(total, wg_ref[...], preferred_element_type=jnp.float32)
        + bg_ref[...] + gf_ref[...])
    x = x_ref[...]
    hm = jnp.tanh(
        jnp.dot(x, wm1x_ref[...], preferred_element_type=jnp.float32)
        + jnp.dot(g, wm1g_ref[...], preferred_element_type=jnp.float32)
        + bm1_ref[...])
    hh = jnp.tanh(
        jnp.dot(x, wh1x_ref[...], preferred_element_type=jnp.float32)
        + jnp.dot(g, wh1g_ref[...], preferred_element_type=jnp.float32)
        + bh1_ref[...])
    ha = jnp.tanh(
        jnp.dot(x, wa1x_ref[...], preferred_element_type=jnp.float32)
        + jnp.dot(g, wa1g_ref[...], preferred_element_type=jnp.float32)
        + ba1_ref[...])
    hall = jnp.concatenate([hm, hh, ha], axis=1)
    o_ref[...] = jnp.tanh(
        jnp.dot(hall, w2c_ref[...], preferred_element_type=jnp.float32)
        + b2c_ref[...])


def _tc_heads(x, psum, gf, wg, bg, wm1x, wm1g, bm1, wh1x, wh1g, bh1,
              wa1x, wa1g, ba1, w2c, b2c, bn):
    n, d = x.shape
    grid = n // bn
    nb = psum.shape[0]
    g_dim = gf.shape[1]
    h_dim = wm1x.shape[1]
    return pl.pallas_call(
        functools.partial(_heads_body, n),
        grid=(grid,),
        in_specs=[
            pl.BlockSpec((bn, d), lambda i: (i, 0)),
            pl.BlockSpec((nb, 1, d), lambda i: (0, 0, 0)),
            pl.BlockSpec((1, g_dim), lambda i: (0, 0)),
            pl.BlockSpec((d, g_dim), lambda i: (0, 0)),
            pl.BlockSpec((1, g_dim), lambda i: (0, 0)),
            pl.BlockSpec((d, h_dim), lambda i: (0, 0)),
            pl.BlockSpec((g_dim, h_dim), lambda i: (0, 0)),
            pl.BlockSpec((1, h_dim), lambda i: (0, 0)),
            pl.BlockSpec((d, h_dim), lambda i: (0, 0)),
            pl.BlockSpec((g_dim, h_dim), lambda i: (0, 0)),
            pl.BlockSpec((1, h_dim), lambda i: (0, 0)),
            pl.BlockSpec((d, h_dim), lambda i: (0, 0)),
            pl.BlockSpec((g_dim, h_dim), lambda i: (0, 0)),
            pl.BlockSpec((1, h_dim), lambda i: (0, 0)),
            pl.BlockSpec((3 * h_dim, 8), lambda i: (0, 0)),
            pl.BlockSpec((1, 8), lambda i: (0, 0)),
        ],
        out_specs=pl.BlockSpec((bn, 8), lambda i: (i, 0)),
        out_shape=jax.ShapeDtypeStruct((n, 8), jnp.float32),
    )(x, psum, gf, wg, bg.reshape(1, g_dim), wm1x, wm1g, bm1.reshape(1, h_dim),
      wh1x, wh1g, bh1.reshape(1, h_dim), wa1x, wa1g, ba1.reshape(1, h_dim),
      w2c, b2c)


def _sc_edge_agg(msg, src, dst, zeros_tile):
    n, d = msg.shape
    e = src.shape[0]
    n_chunks = e // _CHUNK
    n_iters = (n_chunks + _NW - 1) // _NW
    rows_per_tile = ((n // _NS + 7) // 8) * 8
    n_pad = rows_per_tile * _NS
    mesh = plsc.VectorSubcoreMesh(core_axis_name="c", subcore_axis_name="s")

    @functools.partial(
        pl.kernel,
        out_type=jax.ShapeDtypeStruct((_NC * n_pad, d), jnp.float32),
        mesh=mesh,
        scratch_types=[
            pltpu.VMEM((_CHUNK,), jnp.int32),
            pltpu.VMEM((_CHUNK,), jnp.int32),
            pltpu.VMEM((_CHUNK, d), jnp.float32),
            pltpu.VMEM_SHARED((n_pad, d), jnp.float32),
            pltpu.SemaphoreType.DMA,
        ],
        compiler_params=pltpu.CompilerParams(needs_layout_passes=False),
    )
    def k(msg_hbm, src_hbm, dst_hbm, zero_hbm, out_hbm, sidx, didx, rows, acc, sem):
        c = lax.axis_index("c")
        s = lax.axis_index("s")
        wid = s * _NC + c
        pltpu.sync_copy(zero_hbm, acc.at[pl.ds(s * rows_per_tile, rows_per_tile)])
        plsc.subcore_barrier()

        def body(i, carry):
            chunk = wid + i * _NW

            @pl.when(chunk < n_chunks)
            def _():
                base = chunk * _CHUNK
                pltpu.sync_copy(src_hbm.at[pl.ds(base, _CHUNK)], sidx)
                pltpu.sync_copy(dst_hbm.at[pl.ds(base, _CHUNK)], didx)
                pltpu.async_copy(msg_hbm.at[sidx], rows, sem).wait()
                pltpu.sync_copy(rows, acc.at[didx], add=True)

            return carry

        lax.fori_loop(0, n_iters, body, 0)
        plsc.subcore_barrier()
        pltpu.sync_copy(
            acc.at[pl.ds(s * rows_per_tile, rows_per_tile)],
            out_hbm.at[pl.ds(c * n_pad + s * rows_per_tile, rows_per_tile)])

    return k(msg, src, dst, zeros_tile), n_pad


def _sc_heads_gather(table_flat, ally_pad, adst_pad, move_w):
    tn = table_flat.shape[0]
    apad = ally_pad.shape[0]
    epad = adst_pad.shape[0]
    ept = epad // _NW
    mpt = apad * move_w // _NW
    hpt = apad // _NW
    mesh = plsc.VectorSubcoreMesh(core_axis_name="c", subcore_axis_name="s")

    @functools.partial(
        pl.kernel,
        out_type=(
            jax.ShapeDtypeStruct((apad * move_w,), jnp.float32),
            jax.ShapeDtypeStruct((apad,), jnp.float32),
            jax.ShapeDtypeStruct((epad,), jnp.float32),
        ),
        mesh=mesh,
        scratch_types=[
            pltpu.VMEM((tn,), jnp.float32),
            pltpu.VMEM((apad,), jnp.int32),
            pltpu.VMEM((ept,), jnp.int32),
            pltpu.VMEM((mpt,), jnp.float32),
            pltpu.VMEM((hpt,), jnp.float32),
            pltpu.VMEM((ept,), jnp.float32),
        ],
        compiler_params=pltpu.CompilerParams(needs_layout_passes=False),
    )
    def k(tab_hbm, ally_hbm, adst_hbm, mv_hbm, ho_hbm, at_hbm,
          tab_v, ally_v, adst_v, mv_v, ho_v, at_v):
        c = lax.axis_index("c")
        s = lax.axis_index("s")
        wid = s * _NC + c
        pltpu.sync_copy(tab_hbm, tab_v)
        pltpu.sync_copy(ally_hbm, ally_v)
        pltpu.sync_copy(adst_hbm.at[pl.ds(wid * ept, ept)], adst_v)
        iota = lax.iota(jnp.int32, _LANES)

        mbase = wid * mpt
        for kk in range(mpt // _LANES):
            p = jnp.full((_LANES,), mbase + kk * _LANES, jnp.int32) + iota
            j = p // move_w
            cc = p - j * move_w
            a = plsc.load_gather(ally_v, [j])
            v = plsc.load_gather(tab_v, [a * 8 + cc])
            mv_v[pl.ds(kk * _LANES, _LANES)] = v
        pltpu.sync_copy(mv_v, mv_hbm.at[pl.ds(mbase, mpt)])

        hbase = wid * hpt
        for kk in range(hpt // _LANES):
            p = jnp.full((_LANES,), hbase + kk * _LANES, jnp.int32) + iota
            a = plsc.load_gather(ally_v, [p])
            v = plsc.load_gather(tab_v, [a * 8 + move_w])
            ho_v[pl.ds(kk * _LANES, _LANES)] = v
        pltpu.sync_copy(ho_v, ho_hbm.at[pl.ds(hbase, hpt)])

        def abody(kk, carry):
            d16 = adst_v[pl.ds(kk * _LANES, _LANES)]
            v = plsc.load_gather(tab_v, [d16 * 8 + (move_w + 1)])
            at_v[pl.ds(kk * _LANES, _LANES)] = v
            return carry

        lax.fori_loop(0, ept // _LANES, abody, 0)
        pltpu.sync_copy(at_v, at_hbm.at[pl.ds(wid * ept, ept)])

    return k(table_flat, ally_pad, adst_pad)


def kernel(node_feature, global_feature, edge_index, attack_edge_index,
           ally_indices, W_msg, b_msg, W_u1, b_u1, W_u2, b_u2, ln_g, ln_b,
           W_glob, b_glob, W_m1, b_m1, W_m2, b_m2, W_h1, b_h1, W_h2, b_h2,
           W_a1, b_a1, W_a2, b_a2):
    n, d = node_feature.shape
    ea = attack_edge_index.shape[1]
    ally = ally_indices.shape[0]
    move_w = W_m2.shape[1]
    nlayers = W_msg.shape[0]
    bn = 1000

    src = edge_index[0]
    dst = edge_index[1]
    rows_per_tile = ((n // _NS + 7) // 8) * 8
    zeros_tile = jnp.zeros((rows_per_tile, d), jnp.float32)

    x = node_feature
    psum = None
    for l in range(nlayers):
        msg = _tc_msg(x, W_msg[l], b_msg[l], bn)
        aggcat, n_pad = _sc_edge_agg(msg, src, dst, zeros_tile)
        x, psum = _tc_update(
            x, aggcat[:n], aggcat[n_pad:n_pad + n], W_u1[l][:d], W_u1[l][d:], b_u1[l],
            W_u2[l], b_u2[l], ln_g[l], ln_b[l], bn,
            with_psum=(l == nlayers - 1))

    h_dim = W_m1.shape[1]
    w2c = jnp.zeros((3 * h_dim, 8), jnp.float32)
    w2c = w2c.at[0:h_dim, 0:move_w].set(W_m2)
    w2c = w2c.at[h_dim:2 * h_dim, move_w:move_w + 1].set(W_h2)
    w2c = w2c.at[2 * h_dim:, move_w + 1:move_w + 2].set(W_a2)
    b2c = jnp.zeros((1, 8), jnp.float32)
    b2c = b2c.at[0, 0:move_w].set(b_m2)
    b2c = b2c.at[0, move_w].set(b_h2[0])
    b2c = b2c.at[0, move_w + 1].set(b_a2[0])

    out8 = _tc_heads(
        x, psum, global_feature, W_glob, b_glob,
        W_m1[:d], W_m1[d:], b_m1, W_h1[:d], W_h1[d:], b_h1,
        W_a1[:d], W_a1[d:], b_a1, w2c, b2c, bn)

    apad = ((ally + _NW * _LANES - 1) // (_NW * _LANES)) * _NW * _LANES
    epad = ((ea + _NW * _LANES - 1) // (_NW * _LANES)) * _NW * _LANES
    ally_pad = jnp.zeros((apad,), jnp.int32).at[:ally].set(ally_indices)
    adst_pad = jnp.zeros((epad,), jnp.int32).at[:ea].set(attack_edge_index[1])

    mv, ho, at = _sc_heads_gather(out8.reshape(-1), ally_pad, adst_pad, move_w)
    return (mv[:ally * move_w].reshape(ally, move_w),
            ho[:ally].reshape(ally, 1),
            at[:ea])


# speedup vs baseline: 10.9095x; 1.8002x over previous
import functools

import jax
import jax.numpy as jnp
from jax import lax
from jax.experimental import pallas as pl
from jax.experimental.pallas import tpu as pltpu
from jax.experimental.pallas import tpu_sc as plsc

_NC = 2
_NS = 16
_NW = _NC * _NS
_LANES = 16
_CHUNK = 128


def _msg_body(x_ref, w_ref, b_ref, o_ref):
    m = jnp.tanh(
        jnp.dot(x_ref[...], w_ref[...], preferred_element_type=jnp.float32)
        + b_ref[...])
    o_ref[...] = jnp.maximum(m, 0.0)


def _tc_msg(x, w, b, bn):
    n, d = x.shape
    return pl.pallas_call(
        _msg_body,
        grid=(n // bn,),
        in_specs=[
            pl.BlockSpec((bn, d), lambda i: (i, 0)),
            pl.BlockSpec((d, d), lambda i: (0, 0)),
            pl.BlockSpec((1, d), lambda i: (0, 0)),
        ],
        out_specs=pl.BlockSpec((bn, d), lambda i: (i, 0)),
        out_shape=jax.ShapeDtypeStruct((n, d), jnp.float32),
    )(x, w, b.reshape(1, d))


def _upd_body(with_psum, x_ref, a0_ref, a1_ref, w1x_ref, w1a_ref, b1_ref,
              w2_ref, b2_ref, g_ref, be_ref, *out_refs):
    x = x_ref[...]
    agg = a0_ref[...] + a1_ref[...]
    h = jnp.tanh(
        jnp.dot(x, w1x_ref[...], preferred_element_type=jnp.float32)
        + jnp.dot(agg, w1a_ref[...], preferred_element_type=jnp.float32)
        + b1_ref[...])
    h = jnp.dot(h, w2_ref[...], preferred_element_type=jnp.float32) + b2_ref[...]
    mu = jnp.mean(h, axis=-1, keepdims=True)
    c = h - mu
    var = jnp.mean(c * c, axis=-1, keepdims=True)
    xn = g_ref[...] * c * lax.rsqrt(var + 1e-5) + be_ref[...]
    out_refs[0][...] = xn
    if with_psum:
        out_refs[1][...] = jnp.sum(xn, axis=0)[None, None, :]


def _tc_update(x, a0, a1, w1x, w1a, b1, w2, b2, ln_g, ln_b, bn, with_psum):
    n, d = x.shape
    grid = n // bn
    out_shape = [jax.ShapeDtypeStruct((n, d), jnp.float32)]
    out_specs = [pl.BlockSpec((bn, d), lambda i: (i, 0))]
    if with_psum:
        out_shape.append(jax.ShapeDtypeStruct((grid, 1, d), jnp.float32))
        out_specs.append(pl.BlockSpec((1, 1, d), lambda i: (i, 0, 0)))
    res = pl.pallas_call(
        functools.partial(_upd_body, with_psum),
        grid=(grid,),
        in_specs=[
            pl.BlockSpec((bn, d), lambda i: (i, 0)),
            pl.BlockSpec((bn, d), lambda i: (i, 0)),
            pl.BlockSpec((bn, d), lambda i: (i, 0)),
            pl.BlockSpec((d, d), lambda i: (0, 0)),
            pl.BlockSpec((d, d), lambda i: (0, 0)),
            pl.BlockSpec((1, d), lambda i: (0, 0)),
            pl.BlockSpec((d, d), lambda i: (0, 0)),
            pl.BlockSpec((1, d), lambda i: (0, 0)),
            pl.BlockSpec((1, d), lambda i: (0, 0)),
            pl.BlockSpec((1, d), lambda i: (0, 0)),
        ],
        out_specs=out_specs,
        out_shape=out_shape,
    )(x, a0, a1, w1x, w1a, b1.reshape(1, d), w2, b2.reshape(1, d),
      ln_g.reshape(1, d), ln_b.reshape(1, d))
    if with_psum:
        return res[0], res[1]
    return res[0], None


def _heads_body(n_total, x_ref, ps_ref, gf_ref, wg_ref, bg_ref,
                wm1x_ref, wm1g_ref, bm1_ref, wh1x_ref, wh1g_ref, bh1_ref,
                wa1x_ref, wa1g_ref, ba1_ref, w2c_ref, b2c_ref, o_ref):
    total = jnp.sum(ps_ref[...], axis=(0, 1))[None, :] * (1.0 / n_total)
    g = jnp.tanh(
        jnp.dot(total, wg_ref[...], preferred_element_type=jnp.float32)
        + bg_ref[...] + gf_ref[...])
    x = x_ref[...]
    hm = jnp.tanh(
        jnp.dot(x, wm1x_ref[...], preferred_element_type=jnp.float32)
        + jnp.dot(g, wm1g_ref[...], preferred_element_type=jnp.float32)
        + bm1_ref[...])
    hh = jnp.tanh(
        jnp.dot(x, wh1x_ref[...], preferred_element_type=jnp.float32)
        + jnp.dot(g, wh1g_ref[...], preferred_element_type=jnp.float32)
        + bh1_ref[...])
    ha = jnp.tanh(
        jnp.dot(x, wa1x_ref[...], preferred_element_type=jnp.float32)
        + jnp.dot(g, wa1g_ref[...], preferred_element_type=jnp.float32)
        + ba1_ref[...])
    hall = jnp.concatenate([hm, hh, ha], axis=1)
    o_ref[...] = jnp.tanh(
        jnp.dot(hall, w2c_ref[...], preferred_element_type=jnp.float32)
        + b2c_ref[...])


def _tc_heads(x, psum, gf, wg, bg, wm1x, wm1g, bm1, wh1x, wh1g, bh1,
              wa1x, wa1g, ba1, w2c, b2c, bn):
    n, d = x.shape
    grid = n // bn
    nb = psum.shape[0]
    g_dim = gf.shape[1]
    h_dim = wm1x.shape[1]
    return pl.pallas_call(
        functools.partial(_heads_body, n),
        grid=(grid,),
        in_specs=[
            pl.BlockSpec((bn, d), lambda i: (i, 0)),
            pl.BlockSpec((nb, 1, d), lambda i: (0, 0, 0)),
            pl.BlockSpec((1, g_dim), lambda i: (0, 0)),
            pl.BlockSpec((d, g_dim), lambda i: (0, 0)),
            pl.BlockSpec((1, g_dim), lambda i: (0, 0)),
            pl.BlockSpec((d, h_dim), lambda i: (0, 0)),
            pl.BlockSpec((g_dim, h_dim), lambda i: (0, 0)),
            pl.BlockSpec((1, h_dim), lambda i: (0, 0)),
            pl.BlockSpec((d, h_dim), lambda i: (0, 0)),
            pl.BlockSpec((g_dim, h_dim), lambda i: (0, 0)),
            pl.BlockSpec((1, h_dim), lambda i: (0, 0)),
            pl.BlockSpec((d, h_dim), lambda i: (0, 0)),
            pl.BlockSpec((g_dim, h_dim), lambda i: (0, 0)),
            pl.BlockSpec((1, h_dim), lambda i: (0, 0)),
            pl.BlockSpec((3 * h_dim, 8), lambda i: (0, 0)),
            pl.BlockSpec((1, 8), lambda i: (0, 0)),
        ],
        out_specs=pl.BlockSpec((bn, 8), lambda i: (i, 0)),
        out_shape=jax.ShapeDtypeStruct((n, 8), jnp.float32),
    )(x, psum, gf, wg, bg.reshape(1, g_dim), wm1x, wm1g, bm1.reshape(1, h_dim),
      wh1x, wh1g, bh1.reshape(1, h_dim), wa1x, wa1g, ba1.reshape(1, h_dim),
      w2c, b2c)


_NBUF = 2
_IRING = 6


def _sc_edge_agg(msg, src1d, dst1d, zeros_tile):
    n, d = msg.shape
    n_chunks = src1d.shape[0] // _CHUNK
    cpw = n_chunks // _NW
    rows_per_tile = ((n // _NS + 7) // 8) * 8
    n_pad = rows_per_tile * _NS
    mesh = plsc.VectorSubcoreMesh(core_axis_name="c", subcore_axis_name="s")

    @functools.partial(
        pl.kernel,
        out_type=jax.ShapeDtypeStruct((_NC * n_pad, d), jnp.float32),
        mesh=mesh,
        scratch_types=[
            pltpu.VMEM((_IRING, _CHUNK), jnp.int32),
            pltpu.VMEM((_IRING, _CHUNK), jnp.int32),
            pltpu.VMEM((_NBUF, _CHUNK, d), jnp.float32),
            pltpu.VMEM_SHARED((n_pad, d), jnp.float32),
            pltpu.SemaphoreType.DMA((_IRING,)),
            pltpu.SemaphoreType.DMA((_NBUF,)),
            pltpu.SemaphoreType.DMA((_NBUF,)),
        ],
        compiler_params=pltpu.CompilerParams(needs_layout_passes=False),
    )
    def k(msg_hbm, src_hbm, dst_hbm, zero_hbm, out_hbm, sidx, didx, rows, acc,
          isem, gsem, ssem):
        c = lax.axis_index("c")
        s = lax.axis_index("s")
        wid = s * _NC + c
        base0 = wid * cpw * _CHUNK

        def prefetch_idx(t):
            @pl.when(t < cpw)
            def _():
                sl = lax.rem(t, _IRING)
                base = base0 + t * _CHUNK
                pltpu.async_copy(src_hbm.at[pl.ds(base, _CHUNK)],
                                 sidx.at[sl], isem.at[sl])
                pltpu.async_copy(dst_hbm.at[pl.ds(base, _CHUNK)],
                                 didx.at[sl], isem.at[sl])

        prefetch_idx(0)
        prefetch_idx(1)
        pltpu.sync_copy(zero_hbm, acc.at[pl.ds(s * rows_per_tile, rows_per_tile)])
        plsc.subcore_barrier()

        def step(j, carry):
            @pl.when(j < cpw)
            def _():
                b = lax.rem(j, _NBUF)
                sl = lax.rem(j, _IRING)

                @pl.when(j >= _NBUF)
                def _():
                    pltpu.make_async_copy(rows.at[b], acc.at[didx.at[0]],
                                          ssem.at[b]).wait()

                pltpu.make_async_copy(src_hbm.at[pl.ds(0, _CHUNK)],
                                      sidx.at[sl], isem.at[sl]).wait()
                pltpu.make_async_copy(dst_hbm.at[pl.ds(0, _CHUNK)],
                                      didx.at[sl], isem.at[sl]).wait()
                pltpu.async_copy(msg_hbm.at[sidx.at[sl]], rows.at[b],
                                 gsem.at[b])
                prefetch_idx(j + 2)

            jj = j - 1

            @pl.when((jj >= 0) & (jj < cpw))
            def _():
                bb = lax.rem(jj, _NBUF)
                sl2 = lax.rem(jj, _IRING)
                pltpu.make_async_copy(msg_hbm.at[sidx.at[0]], rows.at[bb],
                                      gsem.at[bb]).wait()
                pltpu.async_copy(rows.at[bb], acc.at[didx.at[sl2]],
                                 ssem.at[bb], add=True)

            return carry

        lax.fori_loop(0, cpw + 1, step, 0)
        for b in range(_NBUF):
            pltpu.make_async_copy(rows.at[b], acc.at[didx.at[0]],
                                  ssem.at[b]).wait()
        plsc.subcore_barrier()
        pltpu.sync_copy(
            acc.at[pl.ds(s * rows_per_tile, rows_per_tile)],
            out_hbm.at[pl.ds(c * n_pad + s * rows_per_tile, rows_per_tile)])

    return k(msg, src1d, dst1d, zeros_tile), n_pad


def _sc_heads_gather(table_flat, ally_pad, adst_pad, move_w):
    tn = table_flat.shape[0]
    apad = ally_pad.shape[0]
    epad = adst_pad.shape[0]
    ept = epad // _NW
    mpt = apad * move_w // _NW
    hpt = apad // _NW
    mesh = plsc.VectorSubcoreMesh(core_axis_name="c", subcore_axis_name="s")

    @functools.partial(
        pl.kernel,
        out_type=(
            jax.ShapeDtypeStruct((apad * move_w,), jnp.float32),
            jax.ShapeDtypeStruct((apad,), jnp.float32),
            jax.ShapeDtypeStruct((epad,), jnp.float32),
        ),
        mesh=mesh,
        scratch_types=[
            pltpu.VMEM((tn,), jnp.float32),
            pltpu.VMEM((apad,), jnp.int32),
            pltpu.VMEM((ept,), jnp.int32),
            pltpu.VMEM((mpt,), jnp.float32),
            pltpu.VMEM((hpt,), jnp.float32),
            pltpu.VMEM((ept,), jnp.float32),
        ],
        compiler_params=pltpu.CompilerParams(needs_layout_passes=False),
    )
    def k(tab_hbm, ally_hbm, adst_hbm, mv_hbm, ho_hbm, at_hbm,
          tab_v, ally_v, adst_v, mv_v, ho_v, at_v):
        c = lax.axis_index("c")
        s = lax.axis_index("s")
        wid = s * _NC + c
        pltpu.sync_copy(tab_hbm, tab_v)
        pltpu.sync_copy(ally_hbm, ally_v)
        pltpu.sync_copy(adst_hbm.at[pl.ds(wid * ept, ept)], adst_v)
        iota = lax.iota(jnp.int32, _LANES)

        mbase = wid * mpt
        for kk in range(mpt // _LANES):
            p = jnp.full((_LANES,), mbase + kk * _LANES, jnp.int32) + iota
            j = p // move_w
            cc = p - j * move_w
            a = plsc.load_gather(ally_v, [j])
            v = plsc.load_gather(tab_v, [a * 8 + cc])
            mv_v[pl.ds(kk * _LANES, _LANES)] = v
        pltpu.sync_copy(mv_v, mv_hbm.at[pl.ds(mbase, mpt)])

        hbase = wid * hpt
        for kk in range(hpt // _LANES):
            p = jnp.full((_LANES,), hbase + kk * _LANES, jnp.int32) + iota
            a = plsc.load_gather(ally_v, [p])
            v = plsc.load_gather(tab_v, [a * 8 + move_w])
            ho_v[pl.ds(kk * _LANES, _LANES)] = v
        pltpu.sync_copy(ho_v, ho_hbm.at[pl.ds(hbase, hpt)])

        def abody(kk, carry):
            d16 = adst_v[pl.ds(kk * _LANES, _LANES)]
            v = plsc.load_gather(tab_v, [d16 * 8 + (move_w + 1)])
            at_v[pl.ds(kk * _LANES, _LANES)] = v
            return carry

        lax.fori_loop(0, ept // _LANES, abody, 0)
        pltpu.sync_copy(at_v, at_hbm.at[pl.ds(wid * ept, ept)])

    return k(table_flat, ally_pad, adst_pad)


def kernel(node_feature, global_feature, edge_index, attack_edge_index,
           ally_indices, W_msg, b_msg, W_u1, b_u1, W_u2, b_u2, ln_g, ln_b,
           W_glob, b_glob, W_m1, b_m1, W_m2, b_m2, W_h1, b_h1, W_h2, b_h2,
           W_a1, b_a1, W_a2, b_a2):
    n, d = node_feature.shape
    ea = attack_edge_index.shape[1]
    ally = ally_indices.shape[0]
    move_w = W_m2.shape[1]
    nlayers = W_msg.shape[0]
    bn = 1000

    e = edge_index.shape[1]
    rows_per_tile = ((n // _NS + 7) // 8) * 8
    n_pad = rows_per_tile * _NS
    zeros_tile = jnp.zeros((rows_per_tile, d), jnp.float32)

    e_unit = _NW * _CHUNK * 8
    e_pad = ((e + e_unit - 1) // e_unit) * e_unit
    pad = e_pad - e
    pad_ar = jnp.arange(pad, dtype=jnp.int32)
    src1d = jnp.concatenate([edge_index[0], pad_ar % n])
    dst1d = jnp.concatenate([edge_index[1], n + pad_ar % (n_pad - n)])

    x = node_feature
    psum = None
    for l in range(nlayers):
        msg = _tc_msg(x, W_msg[l], b_msg[l], bn)
        aggcat, n_pad = _sc_edge_agg(msg, src1d, dst1d, zeros_tile)
        x, psum = _tc_update(
            x, aggcat[:n], aggcat[n_pad:n_pad + n], W_u1[l][:d], W_u1[l][d:], b_u1[l],
            W_u2[l], b_u2[l], ln_g[l], ln_b[l], bn,
            with_psum=(l == nlayers - 1))

    h_dim = W_m1.shape[1]
    w2c = jnp.zeros((3 * h_dim, 8), jnp.float32)
    w2c = w2c.at[0:h_dim, 0:move_w].set(W_m2)
    w2c = w2c.at[h_dim:2 * h_dim, move_w:move_w + 1].set(W_h2)
    w2c = w2c.at[2 * h_dim:, move_w + 1:move_w + 2].set(W_a2)
    b2c = jnp.zeros((1, 8), jnp.float32)
    b2c = b2c.at[0, 0:move_w].set(b_m2)
    b2c = b2c.at[0, move_w].set(b_h2[0])
    b2c = b2c.at[0, move_w + 1].set(b_a2[0])

    out8 = _tc_heads(
        x, psum, global_feature, W_glob, b_glob,
        W_m1[:d], W_m1[d:], b_m1, W_h1[:d], W_h1[d:], b_h1,
        W_a1[:d], W_a1[d:], b_a1, w2c, b2c, bn)

    apad = ((ally + _NW * _LANES - 1) // (_NW * _LANES)) * _NW * _LANES
    epad = ((ea + _NW * _LANES - 1) // (_NW * _LANES)) * _NW * _LANES
    ally_pad = jnp.zeros((apad,), jnp.int32).at[:ally].set(ally_indices)
    adst_pad = jnp.zeros((epad,), jnp.int32).at[:ea].set(attack_edge_index[1])

    mv, ho, at = _sc_heads_gather(out8.reshape(-1), ally_pad, adst_pad, move_w)
    return (mv[:ally * move_w].reshape(ally, move_w),
            ho[:ally].reshape(ally, 1),
            at[:ea])
